# baseline (speedup 1.0000x reference)
import jax
import jax.numpy as jnp
from jax import lax
from jax.experimental import pallas as pl
from jax.experimental.pallas import tpu as pltpu
from jax.experimental.pallas import tpu_sc as plsc

NUM_CORES = 2
NUM_SUBCORES = 16
NW = NUM_CORES * NUM_SUBCORES

BATCH = 16384
LEVELS = 7
DIM = 64
VOCAB_TOTAL = 4 + 8 + 16 + 32 + 64 + 128 + 256
ROWS = BATCH * LEVELS
RPW = ROWS // NW
CH = 448
NCH = RPW // CH


def _body(paths_ref, table_ref, out_ref, pbuf, tbuf, obufA, obufB,
          lsem, ssem):
    wid = lax.axis_index("s") * NUM_CORES + lax.axis_index("c")

    c_p = pltpu.async_copy(paths_ref.at[pl.ds(wid * RPW, RPW)], pbuf,
                           lsem.at[0])
    c_t = pltpu.async_copy(table_ref, tbuf, lsem.at[1])
    c_p.wait()
    c_t.wait()

    iota = lax.iota(jnp.int32, 16)
    sevens = jnp.full((16,), LEVELS, jnp.int32)
    fours = jnp.full((16,), 4, jnp.int32)
    obufs = [obufA, obufB]
    s_copy = {}

    for ch in range(NCH):
        ob = obufs[ch % 2]
        if ch >= 2:
            s_copy[ch - 2].wait()

        def fill(i, carry, ch=ch, ob=ob):
            rb = ch * CH + i * 16
            r = lax.rem(rb + iota, sevens)
            iv = pbuf[pl.ds(rb, 16)] + lax.shift_left(fours, r) - 4
            rowv = i * 16 + iota
            for c in range(DIM):
                cc = jnp.full((16,), c, jnp.int32)
                v = plsc.load_gather(tbuf, [iv, cc])
                plsc.store_scatter(ob, [rowv, cc], v)
            return carry

        lax.fori_loop(0, CH // 16, fill, 0)
        s_copy[ch] = pltpu.async_copy(
            ob, out_ref.at[pl.ds(wid * RPW + ch * CH, CH)],
            ssem.at[ch % 2])

    s_copy[NCH - 2].wait()
    s_copy[NCH - 1].wait()


@jax.jit
def kernel(paths, W0, W1, W2, W3, W4, W5, W6):
    table = jnp.concatenate([W0, W1, W2, W3, W4, W5, W6], axis=0)
    paths_flat = paths.reshape(ROWS)

    mesh = plsc.VectorSubcoreMesh(core_axis_name="c", subcore_axis_name="s")
    out = pl.kernel(
        _body,
        out_type=jax.ShapeDtypeStruct((ROWS, DIM), jnp.float32),
        mesh=mesh,
        compiler_params=pltpu.CompilerParams(
            use_tc_tiling_on_sc=False, needs_layout_passes=False),
        scratch_types=[
            pltpu.VMEM((RPW,), jnp.int32),
            pltpu.VMEM((VOCAB_TOTAL, DIM), jnp.float32),
            pltpu.VMEM((CH, DIM), jnp.float32),
            pltpu.VMEM((CH, DIM), jnp.float32),
            pltpu.SemaphoreType.DMA((2,)),
            pltpu.SemaphoreType.DMA((2,)),
        ],
    )(paths_flat, table)
    return out.reshape(BATCH, LEVELS * DIM)


# speedup vs baseline: 2.3274x; 2.3274x over previous
import jax
import jax.numpy as jnp
from jax import lax
from jax.experimental import pallas as pl
from jax.experimental.pallas import tpu as pltpu
from jax.experimental.pallas import tpu_sc as plsc

NUM_CORES = 2
NUM_SUBCORES = 16
NW = NUM_CORES * NUM_SUBCORES

BATCH = 16384
LEVELS = 7
DIM = 64
VOCAB_TOTAL = 4 + 8 + 16 + 32 + 64 + 128 + 256
ROWS = BATCH * LEVELS
RPW = ROWS // NW
CH = 448
NCH = RPW // CH


def _body(paths_ref, table_ref, out_ref, pbuf, tbuf, obufA, obufB,
          lsem, ssem):
    wid = lax.axis_index("s") * NUM_CORES + lax.axis_index("c")

    c_p = pltpu.async_copy(paths_ref.at[pl.ds(wid * RPW, RPW)], pbuf,
                           lsem.at[0])
    c_t = pltpu.async_copy(table_ref, tbuf, lsem.at[1])
    c_p.wait()
    c_t.wait()

    iota = lax.iota(jnp.int32, 16)
    sevens = jnp.full((16,), LEVELS, jnp.int32)
    fours = jnp.full((16,), 4, jnp.int32)
    obufs = [obufA, obufB]
    s_copy = {}

    for ch in range(NCH):
        ob = obufs[ch % 2]
        if ch >= 2:
            s_copy[ch - 2].wait()

        def fill(i, carry, ch=ch, ob=ob):
            rb = ch * CH + i * 16
            r = lax.rem(rb + iota, sevens)
            iv = pbuf[pl.ds(rb, 16)] + lax.shift_left(fours, r) - 4
            for j in range(16):
                ivj = lax.gather(
                    iv, jnp.full((16, 1), j, jnp.int32),
                    dimension_numbers=lax.GatherDimensionNumbers(
                        offset_dims=(), collapsed_slice_dims=(0,),
                        start_index_map=(0,)),
                    slice_sizes=(1,),
                    mode=lax.GatherScatterMode.PROMISE_IN_BOUNDS)
                for g in range(DIM // 16):
                    v = plsc.load_gather(tbuf, [ivj, g * 16 + iota])
                    ob[i * 16 + j, pl.ds(g * 16, 16)] = v
            return carry

        lax.fori_loop(0, CH // 16, fill, 0)
        s_copy[ch] = pltpu.async_copy(
            ob, out_ref.at[pl.ds(wid * RPW + ch * CH, CH)],
            ssem.at[ch % 2])

    s_copy[NCH - 2].wait()
    s_copy[NCH - 1].wait()


@jax.jit
def kernel(paths, W0, W1, W2, W3, W4, W5, W6):
    table = jnp.concatenate([W0, W1, W2, W3, W4, W5, W6], axis=0)
    paths_flat = paths.reshape(ROWS)

    mesh = plsc.VectorSubcoreMesh(core_axis_name="c", subcore_axis_name="s")
    out = pl.kernel(
        _body,
        out_type=jax.ShapeDtypeStruct((ROWS, DIM), jnp.float32),
        mesh=mesh,
        compiler_params=pltpu.CompilerParams(
            use_tc_tiling_on_sc=False, needs_layout_passes=False),
        scratch_types=[
            pltpu.VMEM((RPW,), jnp.int32),
            pltpu.VMEM((VOCAB_TOTAL, DIM), jnp.float32),
            pltpu.VMEM((CH, DIM), jnp.float32),
            pltpu.VMEM((CH, DIM), jnp.float32),
            pltpu.SemaphoreType.DMA((2,)),
            pltpu.SemaphoreType.DMA((2,)),
        ],
    )(paths_flat, table)
    return out.reshape(BATCH, LEVELS * DIM)


# speedup vs baseline: 2.3345x; 1.0031x over previous
import jax
import jax.numpy as jnp
from jax import lax
from jax.experimental import pallas as pl
from jax.experimental.pallas import tpu as pltpu
from jax.experimental.pallas import tpu_sc as plsc

NUM_CORES = 2
NUM_SUBCORES = 16
NW = NUM_CORES * NUM_SUBCORES

BATCH = 16384
LEVELS = 7
DIM = 64
VOCAB_OFFSETS = (0, 4, 12, 28, 60, 124, 252)
VOCAB_TOTAL = 508
ROWS = BATCH * LEVELS
RPW = ROWS // NW
CH = 448
NCH = RPW // CH


def _body(idx_ref, table_ref, out_ref, ibuf, tbuf, obufA, obufB,
          lsem, ssem):
    wid = lax.axis_index("s") * NUM_CORES + lax.axis_index("c")

    c_i = pltpu.async_copy(idx_ref.at[pl.ds(wid * RPW, RPW)], ibuf,
                           lsem.at[0])
    c_t = pltpu.async_copy(table_ref, tbuf, lsem.at[1])
    c_i.wait()
    c_t.wait()

    iota = lax.iota(jnp.int32, 16)
    obufs = [obufA, obufB]
    s_copy = {}

    for ch in range(NCH):
        ob = obufs[ch % 2]
        if ch >= 2:
            s_copy[ch - 2].wait()

        def fill(i, carry, ch=ch, ob=ob):
            rb = ch * CH + i * 16
            iv = ibuf[pl.ds(rb, 16)]
            for j in range(16):
                ivj = lax.gather(
                    iv, jnp.full((16, 1), j, jnp.int32),
                    dimension_numbers=lax.GatherDimensionNumbers(
                        offset_dims=(), collapsed_slice_dims=(0,),
                        start_index_map=(0,)),
                    slice_sizes=(1,),
                    mode=lax.GatherScatterMode.PROMISE_IN_BOUNDS)
                for g in range(DIM // 16):
                    v = plsc.load_gather(tbuf, [ivj, g * 16 + iota])
                    ob[i * 16 + j, pl.ds(g * 16, 16)] = v
            return carry

        lax.fori_loop(0, CH // 16, fill, 0)
        s_copy[ch] = pltpu.async_copy(
            ob, out_ref.at[pl.ds(wid * RPW + ch * CH, CH)],
            ssem.at[ch % 2])

    s_copy[NCH - 2].wait()
    s_copy[NCH - 1].wait()


@jax.jit
def kernel(paths, W0, W1, W2, W3, W4, W5, W6):
    table = jnp.concatenate([W0, W1, W2, W3, W4, W5, W6], axis=0)
    offs = jnp.array(VOCAB_OFFSETS, dtype=jnp.int32)
    idx_flat = (paths + offs[None, :]).reshape(ROWS)

    mesh = plsc.VectorSubcoreMesh(core_axis_name="c", subcore_axis_name="s")
    out = pl.kernel(
        _body,
        out_type=jax.ShapeDtypeStruct((ROWS, DIM), jnp.float32),
        mesh=mesh,
        compiler_params=pltpu.CompilerParams(
            use_tc_tiling_on_sc=False, needs_layout_passes=False),
        scratch_types=[
            pltpu.VMEM((RPW,), jnp.int32),
            pltpu.VMEM((VOCAB_TOTAL, DIM), jnp.float32),
            pltpu.VMEM((CH, DIM), jnp.float32),
            pltpu.VMEM((CH, DIM), jnp.float32),
            pltpu.SemaphoreType.DMA((2,)),
            pltpu.SemaphoreType.DMA((2,)),
        ],
    )(idx_flat, table)
    return out.reshape(BATCH, LEVELS * DIM)
